# Initial kernel scaffold; baseline (speedup 1.0000x reference)
import jax
import jax.numpy as jnp
from jax.experimental import pallas as pl


def kernel(x, edge_index, Wl0, Wr0, b0, g0, be0, Wl1, Wr1, b1, g1, be1, Wl2, Wr2, b2, g2, be2, Wc1, bc1, Wc2, bc2):
    raise NotImplementedError("write your pallas kernel here")


# speedup vs baseline: 7.6641x; 7.6641x over previous
import jax
import jax.numpy as jnp
from jax import lax
from jax.experimental import pallas as pl
from jax.experimental.pallas import tpu as pltpu
from jax.experimental.pallas import tpu_sc as plsc

N = 10000
D = 128
NP = 10112
NC = 2
NS = 16
NW = NC * NS
BATCH = 128
NBATCH = 80
EP = NW * NBATCH * BATCH
STRIPE = NP // NS


def _agg_body(src_hbm, dst_hbm, h_hbm, zrow_hbm, out_hbm,
              acc_sh, srcv, dstv, rows):
    c = lax.axis_index("c")
    s = lax.axis_index("s")
    wid = c * NS + s

    pltpu.sync_copy(src_hbm.at[wid], srcv)
    pltpu.sync_copy(dst_hbm.at[wid], dstv)
    pltpu.sync_copy(zrow_hbm, acc_sh.at[pl.ds(s * STRIPE, STRIPE)])
    plsc.subcore_barrier()

    def step(j, carry):
        pltpu.sync_copy(h_hbm.at[srcv.at[j]], rows)
        pltpu.sync_copy(rows, acc_sh.at[dstv.at[j]], add=True)
        return carry

    lax.fori_loop(0, NBATCH, step, 0)
    plsc.subcore_barrier()

    pltpu.sync_copy(acc_sh.at[pl.ds(s * STRIPE, STRIPE)],
                    out_hbm.at[pl.ds(c * NP + s * STRIPE, STRIPE)])


_agg = pl.kernel(
    _agg_body,
    out_type=jax.ShapeDtypeStruct((NC * NP, D), jnp.float32),
    mesh=plsc.VectorSubcoreMesh(core_axis_name="c", subcore_axis_name="s"),
    scratch_types=[
        pltpu.VMEM_SHARED((NP, D), jnp.float32),
        pltpu.VMEM((NBATCH, BATCH), jnp.int32),
        pltpu.VMEM((NBATCH, BATCH), jnp.int32),
        pltpu.VMEM((BATCH, D), jnp.float32),
    ],
)


def _cnt_body(dst_hbm, zrow_hbm, ones_hbm, out_hbm, cnt_sh, dstv, onesv):
    c = lax.axis_index("c")
    s = lax.axis_index("s")
    wid = c * NS + s

    pltpu.sync_copy(dst_hbm.at[wid], dstv)
    pltpu.sync_copy(zrow_hbm, cnt_sh.at[pl.ds(s * STRIPE, STRIPE)])
    pltpu.sync_copy(ones_hbm, onesv)
    plsc.subcore_barrier()

    def step(j, carry):
        pltpu.sync_copy(onesv, cnt_sh.at[dstv.at[j]], add=True)
        return carry

    lax.fori_loop(0, NBATCH, step, 0)
    plsc.subcore_barrier()

    pltpu.sync_copy(cnt_sh.at[pl.ds(s * STRIPE, STRIPE)],
                    out_hbm.at[pl.ds(c * NP + s * STRIPE, STRIPE)])


_cnt = pl.kernel(
    _cnt_body,
    out_type=jax.ShapeDtypeStruct((NC * NP, D), jnp.float32),
    mesh=plsc.VectorSubcoreMesh(core_axis_name="c", subcore_axis_name="s"),
    scratch_types=[
        pltpu.VMEM_SHARED((NP, D), jnp.float32),
        pltpu.VMEM((NBATCH, BATCH), jnp.int32),
        pltpu.VMEM((BATCH, D), jnp.float32),
    ],
)


def _dense_body(acc_ref, cnt_ref, h_ref, wl_ref, wr_ref, b_ref, g_ref,
                be_ref, out_ref):
    msum = acc_ref[:N, :] + acc_ref[NP:NP + N, :]
    cnt = cnt_ref[:N, 0:1] + cnt_ref[NP:NP + N, 0:1]
    mean = msum * (1.0 / jnp.maximum(cnt, 1.0))
    t = (jnp.dot(mean, wl_ref[:], preferred_element_type=jnp.float32)
         + jnp.dot(h_ref[:], wr_ref[:], preferred_element_type=jnp.float32)
         + b_ref[:])
    mu = jnp.mean(t, axis=0, keepdims=True)
    var = jnp.mean((t - mu) * (t - mu), axis=0, keepdims=True)
    y = (t - mu) * lax.rsqrt(var + 1e-5) * g_ref[:] + be_ref[:]
    out_ref[:] = jnp.maximum(y, 0.0)


_dense = pl.pallas_call(
    _dense_body,
    out_shape=jax.ShapeDtypeStruct((N, D), jnp.float32),
)


def _mlp_body(h1_ref, h2_ref, h3_ref, w1a_ref, w1b_ref, w1c_ref, bc1_ref,
              w2_ref, bc2_ref, out_ref):
    z = (jnp.dot(h1_ref[:], w1a_ref[:], preferred_element_type=jnp.float32)
         + jnp.dot(h2_ref[:], w1b_ref[:], preferred_element_type=jnp.float32)
         + jnp.dot(h3_ref[:], w1c_ref[:], preferred_element_type=jnp.float32)
         + bc1_ref[:])
    z = jnp.maximum(z, 0.0)
    out_ref[:] = (jnp.dot(z, w2_ref[:], preferred_element_type=jnp.float32)
                  + bc2_ref[:])


_mlp = pl.pallas_call(
    _mlp_body,
    out_shape=jax.ShapeDtypeStruct((N, 128), jnp.float32),
)


def kernel(x, edge_index, Wl0, Wr0, b0, g0, be0, Wl1, Wr1, b1, g1, be1,
           Wl2, Wr2, b2, g2, be2, Wc1, bc1, Wc2, bc2):
    src = edge_index[0]
    dst = edge_index[1]
    pad = EP - src.shape[0]
    ar = jnp.arange(pad, dtype=jnp.int32)
    src_p = jnp.concatenate([src, ar % N]).reshape(NW, NBATCH, BATCH)
    dst_p = jnp.concatenate([dst, N + (ar % 16)]).reshape(NW, NBATCH, BATCH)

    zrow = jnp.zeros((STRIPE, D), jnp.float32)
    ones = jnp.ones((BATCH, D), jnp.float32)

    cnt = _cnt(dst_p, zrow, ones)
    acc0 = _agg(src_p, dst_p, x, zrow)

    b0r, g0r, be0r = b0[None, :], g0[None, :], be0[None, :]
    b1r, g1r, be1r = b1[None, :], g1[None, :], be1[None, :]
    b2r, g2r, be2r = b2[None, :], g2[None, :], be2[None, :]

    h1 = _dense(acc0, cnt, x, Wl0, Wr0, b0r, g0r, be0r)
    acc1 = _agg(src_p, dst_p, h1, zrow)
    h2 = _dense(acc1, cnt, h1, Wl1, Wr1, b1r, g1r, be1r)
    acc2 = _agg(src_p, dst_p, h2, zrow)
    h3 = _dense(acc2, cnt, h2, Wl2, Wr2, b2r, g2r, be2r)

    w1a, w1b, w1c = Wc1[:128], Wc1[128:256], Wc1[256:]
    w2p = jnp.zeros((64, 128), jnp.float32).at[:, :2].set(Wc2)
    bc2p = jnp.zeros((1, 128), jnp.float32).at[0, :2].set(bc2)
    out = _mlp(h1, h2, h3, w1a, w1b, w1c, bc1[None, :], w2p, bc2p)
    return out[:, :2]


# speedup vs baseline: 10.7000x; 1.3961x over previous
import jax
import jax.numpy as jnp
from jax import lax
from jax.experimental import pallas as pl
from jax.experimental.pallas import tpu as pltpu
from jax.experimental.pallas import tpu_sc as plsc

N = 10000
D = 128
NP = 10112
NC = 2
NS = 16
NW = NC * NS
BATCH = 128
NBATCH = 80
EP = NW * NBATCH * BATCH
STRIPE = NP // NS


HB = NBATCH // 2


def _agg_body(src_hbm, dst_hbm, h_hbm, zrow_hbm, out_hbm,
              acc_sh, srcv, dstv, rows0, rows1, semA, semB):
    c = lax.axis_index("c")
    s = lax.axis_index("s")
    wid = c * NS + s

    pltpu.sync_copy(zrow_hbm, acc_sh.at[pl.ds(s * STRIPE, STRIPE)])
    plsc.subcore_barrier()

    for half in range(2):
        pltpu.sync_copy(src_hbm.at[wid, pl.ds(half * HB, HB)], srcv)
        pltpu.sync_copy(dst_hbm.at[wid, pl.ds(half * HB, HB)], dstv)
        pltpu.async_copy(h_hbm.at[srcv.at[0]], rows0, semA)

        def step(jj, carry):
            j0 = 2 * jj
            j1 = j0 + 1
            pltpu.async_copy(h_hbm.at[srcv.at[j1]], rows1, semB)
            pltpu.make_async_copy(h_hbm.at[srcv.at[j0]], rows0, semA).wait()
            pltpu.sync_copy(rows0, acc_sh.at[dstv.at[j0]], add=True)

            @pl.when(j0 + 2 < HB)
            def _():
                pltpu.async_copy(h_hbm.at[srcv.at[j0 + 2]], rows0, semA)

            pltpu.make_async_copy(h_hbm.at[srcv.at[j1]], rows1, semB).wait()
            pltpu.sync_copy(rows1, acc_sh.at[dstv.at[j1]], add=True)
            return carry

        lax.fori_loop(0, HB // 2, step, 0)
    plsc.subcore_barrier()

    pltpu.sync_copy(acc_sh.at[pl.ds(s * STRIPE, STRIPE)],
                    out_hbm.at[pl.ds(c * NP + s * STRIPE, STRIPE)])


_agg = pl.kernel(
    _agg_body,
    out_type=jax.ShapeDtypeStruct((NC * NP, D), jnp.float32),
    mesh=plsc.VectorSubcoreMesh(core_axis_name="c", subcore_axis_name="s"),
    scratch_types=[
        pltpu.VMEM_SHARED((NP, D), jnp.float32),
        pltpu.VMEM((HB, BATCH), jnp.int32),
        pltpu.VMEM((HB, BATCH), jnp.int32),
        pltpu.VMEM((BATCH, D), jnp.float32),
        pltpu.VMEM((BATCH, D), jnp.float32),
        pltpu.SemaphoreType.DMA,
        pltpu.SemaphoreType.DMA,
    ],
)


def _cnt_body(dst_hbm, zrow_hbm, ones_hbm, out_hbm, cnt_sh, dstv, onesv,
              sem):
    c = lax.axis_index("c")
    s = lax.axis_index("s")
    wid = c * NS + s

    pltpu.sync_copy(dst_hbm.at[wid], dstv)
    pltpu.sync_copy(zrow_hbm, cnt_sh.at[pl.ds(s * STRIPE, STRIPE)])
    pltpu.sync_copy(ones_hbm, onesv)
    plsc.subcore_barrier()

    def step(jj, carry):
        descs = [pltpu.async_copy(onesv, cnt_sh.at[dstv.at[4 * jj + k]],
                                  sem, add=True)
                 for k in range(4)]
        for d in descs:
            d.wait()
        return carry

    lax.fori_loop(0, NBATCH // 4, step, 0)
    plsc.subcore_barrier()

    pltpu.sync_copy(cnt_sh.at[pl.ds(s * STRIPE, STRIPE)],
                    out_hbm.at[pl.ds(c * NP + s * STRIPE, STRIPE)])


_cnt = pl.kernel(
    _cnt_body,
    out_type=jax.ShapeDtypeStruct((NC * NP, D), jnp.float32),
    mesh=plsc.VectorSubcoreMesh(core_axis_name="c", subcore_axis_name="s"),
    scratch_types=[
        pltpu.VMEM_SHARED((NP, D), jnp.float32),
        pltpu.VMEM((NBATCH, BATCH), jnp.int32),
        pltpu.VMEM((BATCH, D), jnp.float32),
        pltpu.SemaphoreType.DMA,
    ],
)


def _dense_body(acc_ref, cnt_ref, h_ref, wl_ref, wr_ref, b_ref, g_ref,
                be_ref, out_ref):
    msum = acc_ref[:N, :] + acc_ref[NP:NP + N, :]
    cnt = cnt_ref[:N, 0:1] + cnt_ref[NP:NP + N, 0:1]
    mean = msum * (1.0 / jnp.maximum(cnt, 1.0))
    t = (jnp.dot(mean, wl_ref[:], preferred_element_type=jnp.float32)
         + jnp.dot(h_ref[:], wr_ref[:], preferred_element_type=jnp.float32)
         + b_ref[:])
    mu = jnp.mean(t, axis=0, keepdims=True)
    var = jnp.mean((t - mu) * (t - mu), axis=0, keepdims=True)
    y = (t - mu) * lax.rsqrt(var + 1e-5) * g_ref[:] + be_ref[:]
    out_ref[:] = jnp.maximum(y, 0.0)


_dense = pl.pallas_call(
    _dense_body,
    out_shape=jax.ShapeDtypeStruct((N, D), jnp.float32),
)


def _mlp_body(h1_ref, h2_ref, h3_ref, w1a_ref, w1b_ref, w1c_ref, bc1_ref,
              w2_ref, bc2_ref, out_ref):
    z = (jnp.dot(h1_ref[:], w1a_ref[:], preferred_element_type=jnp.float32)
         + jnp.dot(h2_ref[:], w1b_ref[:], preferred_element_type=jnp.float32)
         + jnp.dot(h3_ref[:], w1c_ref[:], preferred_element_type=jnp.float32)
         + bc1_ref[:])
    z = jnp.maximum(z, 0.0)
    out_ref[:] = (jnp.dot(z, w2_ref[:], preferred_element_type=jnp.float32)
                  + bc2_ref[:])


_mlp = pl.pallas_call(
    _mlp_body,
    out_shape=jax.ShapeDtypeStruct((N, 128), jnp.float32),
)


def kernel(x, edge_index, Wl0, Wr0, b0, g0, be0, Wl1, Wr1, b1, g1, be1,
           Wl2, Wr2, b2, g2, be2, Wc1, bc1, Wc2, bc2):
    src = edge_index[0]
    dst = edge_index[1]
    pad = EP - src.shape[0]
    ar = jnp.arange(pad, dtype=jnp.int32)
    src_p = jnp.concatenate([src, ar % N]).reshape(NW, NBATCH, BATCH)
    dst_p = jnp.concatenate([dst, N + (ar % 16)]).reshape(NW, NBATCH, BATCH)

    zrow = jnp.zeros((STRIPE, D), jnp.float32)
    ones = jnp.ones((BATCH, D), jnp.float32)

    cnt = _cnt(dst_p, zrow, ones)
    acc0 = _agg(src_p, dst_p, x, zrow)

    b0r, g0r, be0r = b0[None, :], g0[None, :], be0[None, :]
    b1r, g1r, be1r = b1[None, :], g1[None, :], be1[None, :]
    b2r, g2r, be2r = b2[None, :], g2[None, :], be2[None, :]

    h1 = _dense(acc0, cnt, x, Wl0, Wr0, b0r, g0r, be0r)
    acc1 = _agg(src_p, dst_p, h1, zrow)
    h2 = _dense(acc1, cnt, h1, Wl1, Wr1, b1r, g1r, be1r)
    acc2 = _agg(src_p, dst_p, h2, zrow)
    h3 = _dense(acc2, cnt, h2, Wl2, Wr2, b2r, g2r, be2r)

    w1a, w1b, w1c = Wc1[:128], Wc1[128:256], Wc1[256:]
    w2p = jnp.zeros((64, 128), jnp.float32).at[:, :2].set(Wc2)
    bc2p = jnp.zeros((1, 128), jnp.float32).at[0, :2].set(bc2)
    out = _mlp(h1, h2, h3, w1a, w1b, w1c, bc1[None, :], w2p, bc2p)
    return out[:, :2]


# speedup vs baseline: 10.8392x; 1.0130x over previous
import jax
import jax.numpy as jnp
from jax import lax
from jax.experimental import pallas as pl
from jax.experimental.pallas import tpu as pltpu
from jax.experimental.pallas import tpu_sc as plsc

N = 10000
D = 128
NP = 10112
NC = 2
NS = 16
NW = NC * NS
BATCH = 128
NBATCH = 80
EP = NW * NBATCH * BATCH
STRIPE = NP // NS


HB = NBATCH // 2


def _agg_body(src_hbm, dst_hbm, h_hbm, zrow_hbm, out_hbm,
              acc_sh, srcv, dstv, rows0, rows1, semA, semB):
    c = lax.axis_index("c")
    s = lax.axis_index("s")
    wid = c * NS + s

    pltpu.sync_copy(zrow_hbm, acc_sh.at[pl.ds(s * STRIPE, STRIPE)])
    plsc.subcore_barrier()

    for half in range(2):
        pltpu.sync_copy(src_hbm.at[wid, pl.ds(half * HB, HB)], srcv)
        pltpu.sync_copy(dst_hbm.at[wid, pl.ds(half * HB, HB)], dstv)
        pltpu.async_copy(h_hbm.at[srcv.at[0]], rows0, semA)

        def step(jj, carry):
            j0 = 2 * jj
            j1 = j0 + 1
            pltpu.async_copy(h_hbm.at[srcv.at[j1]], rows1, semB)
            pltpu.make_async_copy(h_hbm.at[srcv.at[j0]], rows0, semA).wait()
            pltpu.sync_copy(rows0, acc_sh.at[dstv.at[j0]], add=True)

            @pl.when(j0 + 2 < HB)
            def _():
                pltpu.async_copy(h_hbm.at[srcv.at[j0 + 2]], rows0, semA)

            pltpu.make_async_copy(h_hbm.at[srcv.at[j1]], rows1, semB).wait()
            pltpu.sync_copy(rows1, acc_sh.at[dstv.at[j1]], add=True)
            return carry

        lax.fori_loop(0, HB // 2, step, 0)
    plsc.subcore_barrier()

    pltpu.sync_copy(acc_sh.at[pl.ds(s * STRIPE, STRIPE)],
                    out_hbm.at[pl.ds(c * NP + s * STRIPE, STRIPE)])


_agg = pl.kernel(
    _agg_body,
    out_type=jax.ShapeDtypeStruct((NC * NP, D), jnp.float32),
    mesh=plsc.VectorSubcoreMesh(core_axis_name="c", subcore_axis_name="s"),
    scratch_types=[
        pltpu.VMEM_SHARED((NP, D), jnp.float32),
        pltpu.VMEM((HB, BATCH), jnp.int32),
        pltpu.VMEM((HB, BATCH), jnp.int32),
        pltpu.VMEM((BATCH, D), jnp.float32),
        pltpu.VMEM((BATCH, D), jnp.float32),
        pltpu.SemaphoreType.DMA,
        pltpu.SemaphoreType.DMA,
    ],
)


def _cnt_body(dst_hbm, zrow_hbm, ones_hbm, out_hbm, cnt_sh, dstv, onesv,
              sem):
    c = lax.axis_index("c")
    s = lax.axis_index("s")
    wid = c * NS + s

    pltpu.sync_copy(dst_hbm.at[wid], dstv)
    pltpu.sync_copy(zrow_hbm, cnt_sh.at[pl.ds(s * STRIPE, STRIPE)])
    pltpu.sync_copy(ones_hbm, onesv)
    plsc.subcore_barrier()

    def step(jj, carry):
        descs = [pltpu.async_copy(onesv, cnt_sh.at[dstv.at[4 * jj + k]],
                                  sem, add=True)
                 for k in range(4)]
        for d in descs:
            d.wait()
        return carry

    lax.fori_loop(0, NBATCH // 4, step, 0)
    plsc.subcore_barrier()

    pltpu.sync_copy(cnt_sh.at[pl.ds(s * STRIPE, STRIPE)],
                    out_hbm.at[pl.ds(c * NP + s * STRIPE, STRIPE)])


_cnt = pl.kernel(
    _cnt_body,
    out_type=jax.ShapeDtypeStruct((NC * NP, D), jnp.float32),
    mesh=plsc.VectorSubcoreMesh(core_axis_name="c", subcore_axis_name="s"),
    scratch_types=[
        pltpu.VMEM_SHARED((NP, D), jnp.float32),
        pltpu.VMEM((NBATCH, BATCH), jnp.int32),
        pltpu.VMEM((BATCH, D), jnp.float32),
        pltpu.SemaphoreType.DMA,
    ],
)


def _dense_body(acc_ref, cnt_ref, h_ref, wl_ref, wr_ref, b_ref, g_ref,
                be_ref, out_ref):
    msum = acc_ref[:N, :] + acc_ref[NP:NP + N, :]
    cnt = cnt_ref[:N, 0:1] + cnt_ref[NP:NP + N, 0:1]
    mean = msum * (1.0 / jnp.maximum(cnt, 1.0))
    t = (jnp.dot(mean, wl_ref[:], preferred_element_type=jnp.float32)
         + jnp.dot(h_ref[:], wr_ref[:], preferred_element_type=jnp.float32)
         + b_ref[:])
    mu = jnp.mean(t, axis=0, keepdims=True)
    var = jnp.mean((t - mu) * (t - mu), axis=0, keepdims=True)
    y = (t - mu) * lax.rsqrt(var + 1e-5) * g_ref[:] + be_ref[:]
    out_ref[:] = jnp.maximum(y, 0.0)


_dense = pl.pallas_call(
    _dense_body,
    out_shape=jax.ShapeDtypeStruct((N, D), jnp.float32),
)


def _dense_mlp_body(acc_ref, cnt_ref, h_ref, wl_ref, wr_ref, b_ref, g_ref,
                    be_ref, h1_ref, w1a_ref, w1b_ref, w1c_ref, bc1_ref,
                    w2_ref, bc2_ref, out_ref):
    msum = acc_ref[:N, :] + acc_ref[NP:NP + N, :]
    cnt = cnt_ref[:N, 0:1] + cnt_ref[NP:NP + N, 0:1]
    mean = msum * (1.0 / jnp.maximum(cnt, 1.0))
    t = (jnp.dot(mean, wl_ref[:], preferred_element_type=jnp.float32)
         + jnp.dot(h_ref[:], wr_ref[:], preferred_element_type=jnp.float32)
         + b_ref[:])
    mu = jnp.mean(t, axis=0, keepdims=True)
    var = jnp.mean((t - mu) * (t - mu), axis=0, keepdims=True)
    y = (t - mu) * lax.rsqrt(var + 1e-5) * g_ref[:] + be_ref[:]
    h3 = jnp.maximum(y, 0.0)
    z = (jnp.dot(h1_ref[:], w1a_ref[:], preferred_element_type=jnp.float32)
         + jnp.dot(h_ref[:], w1b_ref[:], preferred_element_type=jnp.float32)
         + jnp.dot(h3, w1c_ref[:], preferred_element_type=jnp.float32)
         + bc1_ref[:])
    z = jnp.maximum(z, 0.0)
    out_ref[:] = (jnp.dot(z, w2_ref[:], preferred_element_type=jnp.float32)
                  + bc2_ref[:])


_dense_mlp = pl.pallas_call(
    _dense_mlp_body,
    out_shape=jax.ShapeDtypeStruct((N, 128), jnp.float32),
)


def kernel(x, edge_index, Wl0, Wr0, b0, g0, be0, Wl1, Wr1, b1, g1, be1,
           Wl2, Wr2, b2, g2, be2, Wc1, bc1, Wc2, bc2):
    src = edge_index[0]
    dst = edge_index[1]
    E = src.shape[0]
    pad = EP - E
    ar = jnp.arange(pad, dtype=jnp.int32)
    psrc = (ar % N).reshape(NW, pad // NW)
    pdst = (N + (ar % 16)).reshape(NW, pad // NW)
    src_p = jnp.concatenate([src.reshape(NW, E // NW), psrc],
                            axis=1).reshape(NW, NBATCH, BATCH)
    dst_p = jnp.concatenate([dst.reshape(NW, E // NW), pdst],
                            axis=1).reshape(NW, NBATCH, BATCH)

    zrow = jnp.zeros((STRIPE, D), jnp.float32)
    ones = jnp.ones((BATCH, D), jnp.float32)

    cnt = _cnt(dst_p, zrow, ones)
    acc0 = _agg(src_p, dst_p, x, zrow)

    b0r, g0r, be0r = b0[None, :], g0[None, :], be0[None, :]
    b1r, g1r, be1r = b1[None, :], g1[None, :], be1[None, :]
    b2r, g2r, be2r = b2[None, :], g2[None, :], be2[None, :]

    h1 = _dense(acc0, cnt, x, Wl0, Wr0, b0r, g0r, be0r)
    acc1 = _agg(src_p, dst_p, h1, zrow)
    h2 = _dense(acc1, cnt, h1, Wl1, Wr1, b1r, g1r, be1r)
    acc2 = _agg(src_p, dst_p, h2, zrow)

    w1a, w1b, w1c = Wc1[:128], Wc1[128:256], Wc1[256:]
    w2p = jnp.zeros((64, 128), jnp.float32).at[:, :2].set(Wc2)
    bc2p = jnp.zeros((1, 128), jnp.float32).at[0, :2].set(bc2)
    out = _dense_mlp(acc2, cnt, h2, Wl2, Wr2, b2r, g2r, be2r,
                     h1, w1a, w1b, w1c, bc1[None, :], w2p, bc2p)
    return out[:, :2]


# speedup vs baseline: 12.5713x; 1.1598x over previous
import jax
import jax.numpy as jnp
from jax import lax
from jax.experimental import pallas as pl
from jax.experimental.pallas import tpu as pltpu
from jax.experimental.pallas import tpu_sc as plsc

N = 10000
D = 128
NP = 10112
NC = 2
NS = 16
NW = NC * NS
BATCH = 128
NBATCH = 80
EP = NW * NBATCH * BATCH
STRIPE = NP // NS


HB = NBATCH // 2


def _agg_body(src_hbm, dst_hbm, h_hbm, zrow_hbm, out_hbm,
              acc_sh, srcv, dstv, rows0, rows1, semA, semB):
    c = lax.axis_index("c")
    s = lax.axis_index("s")
    wid = c * NS + s

    pltpu.sync_copy(zrow_hbm, acc_sh.at[pl.ds(s * STRIPE, STRIPE)])
    plsc.subcore_barrier()

    for half in range(2):
        pltpu.sync_copy(src_hbm.at[wid, pl.ds(half * HB, HB)], srcv)
        pltpu.sync_copy(dst_hbm.at[wid, pl.ds(half * HB, HB)], dstv)
        pltpu.async_copy(h_hbm.at[srcv.at[0]], rows0, semA)

        def step(jj, carry):
            j0 = 2 * jj
            j1 = j0 + 1
            pltpu.async_copy(h_hbm.at[srcv.at[j1]], rows1, semB)
            pltpu.make_async_copy(h_hbm.at[srcv.at[j0]], rows0, semA).wait()
            pltpu.sync_copy(rows0, acc_sh.at[dstv.at[j0]], add=True)

            @pl.when(j0 + 2 < HB)
            def _():
                pltpu.async_copy(h_hbm.at[srcv.at[j0 + 2]], rows0, semA)

            pltpu.make_async_copy(h_hbm.at[srcv.at[j1]], rows1, semB).wait()
            pltpu.sync_copy(rows1, acc_sh.at[dstv.at[j1]], add=True)
            return carry

        lax.fori_loop(0, HB // 2, step, 0)
    plsc.subcore_barrier()

    pltpu.sync_copy(acc_sh.at[pl.ds(s * STRIPE, STRIPE)],
                    out_hbm.at[pl.ds(c * NP + s * STRIPE, STRIPE)])


_agg = pl.kernel(
    _agg_body,
    out_type=jax.ShapeDtypeStruct((NC * NP, D), jnp.float32),
    mesh=plsc.VectorSubcoreMesh(core_axis_name="c", subcore_axis_name="s"),
    scratch_types=[
        pltpu.VMEM_SHARED((NP, D), jnp.float32),
        pltpu.VMEM((HB, BATCH), jnp.int32),
        pltpu.VMEM((HB, BATCH), jnp.int32),
        pltpu.VMEM((BATCH, D), jnp.float32),
        pltpu.VMEM((BATCH, D), jnp.float32),
        pltpu.SemaphoreType.DMA,
        pltpu.SemaphoreType.DMA,
    ],
)


def _agg_cnt_body(src_hbm, dst_hbm, h_hbm, zrow_hbm, zc_hbm, out_hbm,
                  cnt_hbm, acc_sh, cnt_sh, srcv, dstv, rows0, rows1, onesv,
                  semA, semB, semC):
    c = lax.axis_index("c")
    s = lax.axis_index("s")
    wid = c * NS + s

    pltpu.sync_copy(zrow_hbm, acc_sh.at[pl.ds(s * STRIPE, STRIPE)])

    @pl.when(s == 0)
    def _():
        pltpu.sync_copy(zc_hbm, cnt_sh)

    def fill(i, carry):
        onesv[pl.ds(i * 16, 16)] = jnp.ones((16,), jnp.float32)
        return carry

    lax.fori_loop(0, BATCH // 16, fill, 0)
    plsc.subcore_barrier()

    for half in range(2):
        pltpu.sync_copy(src_hbm.at[wid, pl.ds(half * HB, HB)], srcv)
        pltpu.sync_copy(dst_hbm.at[wid, pl.ds(half * HB, HB)], dstv)
        pltpu.async_copy(h_hbm.at[srcv.at[0]], rows0, semA)

        def step(jj, carry):
            j0 = 2 * jj
            j1 = j0 + 1
            dc0 = pltpu.async_copy(onesv, cnt_sh.at[dstv.at[j0]], semC,
                                   add=True)
            dc1 = pltpu.async_copy(onesv, cnt_sh.at[dstv.at[j1]], semC,
                                   add=True)
            pltpu.async_copy(h_hbm.at[srcv.at[j1]], rows1, semB)
            pltpu.make_async_copy(h_hbm.at[srcv.at[j0]], rows0, semA).wait()
            pltpu.sync_copy(rows0, acc_sh.at[dstv.at[j0]], add=True)

            @pl.when(j0 + 2 < HB)
            def _():
                pltpu.async_copy(h_hbm.at[srcv.at[j0 + 2]], rows0, semA)

            pltpu.make_async_copy(h_hbm.at[srcv.at[j1]], rows1, semB).wait()
            pltpu.sync_copy(rows1, acc_sh.at[dstv.at[j1]], add=True)
            dc0.wait()
            dc1.wait()
            return carry

        lax.fori_loop(0, HB // 2, step, 0)
    plsc.subcore_barrier()

    pltpu.sync_copy(acc_sh.at[pl.ds(s * STRIPE, STRIPE)],
                    out_hbm.at[pl.ds(c * NP + s * STRIPE, STRIPE)])

    @pl.when(s == 1)
    def _():
        pltpu.sync_copy(cnt_sh, cnt_hbm.at[pl.ds(c * NP, NP)])


_agg_cnt = pl.kernel(
    _agg_cnt_body,
    out_type=(jax.ShapeDtypeStruct((NC * NP, D), jnp.float32),
              jax.ShapeDtypeStruct((NC * NP,), jnp.float32)),
    mesh=plsc.VectorSubcoreMesh(core_axis_name="c", subcore_axis_name="s"),
    scratch_types=[
        pltpu.VMEM_SHARED((NP, D), jnp.float32),
        pltpu.VMEM_SHARED((NP,), jnp.float32),
        pltpu.VMEM((HB, BATCH), jnp.int32),
        pltpu.VMEM((HB, BATCH), jnp.int32),
        pltpu.VMEM((BATCH, D), jnp.float32),
        pltpu.VMEM((BATCH, D), jnp.float32),
        pltpu.VMEM((BATCH,), jnp.float32),
        pltpu.SemaphoreType.DMA,
        pltpu.SemaphoreType.DMA,
        pltpu.SemaphoreType.DMA,
    ],
)


def _dense_body(acc_ref, cnt_ref, h_ref, wl_ref, wr_ref, b_ref, g_ref,
                be_ref, out_ref):
    msum = acc_ref[:N, :] + acc_ref[NP:NP + N, :]
    mean = msum * (1.0 / jnp.maximum(cnt_ref[:], 1.0))
    t = (jnp.dot(mean, wl_ref[:], preferred_element_type=jnp.float32)
         + jnp.dot(h_ref[:], wr_ref[:], preferred_element_type=jnp.float32)
         + b_ref[:])
    mu = jnp.mean(t, axis=0, keepdims=True)
    var = jnp.mean((t - mu) * (t - mu), axis=0, keepdims=True)
    y = (t - mu) * lax.rsqrt(var + 1e-5) * g_ref[:] + be_ref[:]
    out_ref[:] = jnp.maximum(y, 0.0)


_dense = pl.pallas_call(
    _dense_body,
    out_shape=jax.ShapeDtypeStruct((N, D), jnp.float32),
)


def _dense_mlp_body(acc_ref, cnt_ref, h_ref, wl_ref, wr_ref, b_ref, g_ref,
                    be_ref, h1_ref, w1a_ref, w1b_ref, w1c_ref, bc1_ref,
                    w2_ref, bc2_ref, out_ref):
    msum = acc_ref[:N, :] + acc_ref[NP:NP + N, :]
    mean = msum * (1.0 / jnp.maximum(cnt_ref[:], 1.0))
    t = (jnp.dot(mean, wl_ref[:], preferred_element_type=jnp.float32)
         + jnp.dot(h_ref[:], wr_ref[:], preferred_element_type=jnp.float32)
         + b_ref[:])
    mu = jnp.mean(t, axis=0, keepdims=True)
    var = jnp.mean((t - mu) * (t - mu), axis=0, keepdims=True)
    y = (t - mu) * lax.rsqrt(var + 1e-5) * g_ref[:] + be_ref[:]
    h3 = jnp.maximum(y, 0.0)
    z = (jnp.dot(h1_ref[:], w1a_ref[:], preferred_element_type=jnp.float32)
         + jnp.dot(h_ref[:], w1b_ref[:], preferred_element_type=jnp.float32)
         + jnp.dot(h3, w1c_ref[:], preferred_element_type=jnp.float32)
         + bc1_ref[:])
    z = jnp.maximum(z, 0.0)
    out_ref[:] = (jnp.dot(z, w2_ref[:], preferred_element_type=jnp.float32)
                  + bc2_ref[:])


_dense_mlp = pl.pallas_call(
    _dense_mlp_body,
    out_shape=jax.ShapeDtypeStruct((N, 128), jnp.float32),
)


def kernel(x, edge_index, Wl0, Wr0, b0, g0, be0, Wl1, Wr1, b1, g1, be1,
           Wl2, Wr2, b2, g2, be2, Wc1, bc1, Wc2, bc2):
    src = edge_index[0]
    dst = edge_index[1]
    E = src.shape[0]
    pad = EP - E
    ar = jnp.arange(pad, dtype=jnp.int32)
    psrc = (ar % N).reshape(NW, pad // NW)
    pdst = (N + (ar % 16)).reshape(NW, pad // NW)
    src_p = jnp.concatenate([src.reshape(NW, E // NW), psrc],
                            axis=1).reshape(NW, NBATCH, BATCH)
    dst_p = jnp.concatenate([dst.reshape(NW, E // NW), pdst],
                            axis=1).reshape(NW, NBATCH, BATCH)

    zrow = jnp.zeros((STRIPE, D), jnp.float32)
    zc = jnp.zeros((NP,), jnp.float32)

    acc0, cnt1d = _agg_cnt(src_p, dst_p, x, zrow, zc)
    cnt = jnp.broadcast_to((cnt1d[:N] + cnt1d[NP:NP + N])[:, None], (N, D))

    b0r, g0r, be0r = b0[None, :], g0[None, :], be0[None, :]
    b1r, g1r, be1r = b1[None, :], g1[None, :], be1[None, :]
    b2r, g2r, be2r = b2[None, :], g2[None, :], be2[None, :]

    h1 = _dense(acc0, cnt, x, Wl0, Wr0, b0r, g0r, be0r)
    acc1 = _agg(src_p, dst_p, h1, zrow)
    h2 = _dense(acc1, cnt, h1, Wl1, Wr1, b1r, g1r, be1r)
    acc2 = _agg(src_p, dst_p, h2, zrow)

    w1a, w1b, w1c = Wc1[:128], Wc1[128:256], Wc1[256:]
    w2p = jnp.zeros((64, 128), jnp.float32).at[:, :2].set(Wc2)
    bc2p = jnp.zeros((1, 128), jnp.float32).at[0, :2].set(bc2)
    out = _dense_mlp(acc2, cnt, h2, Wl2, Wr2, b2r, g2r, be2r,
                     h1, w1a, w1b, w1c, bc1[None, :], w2p, bc2p)
    return out[:, :2]


# speedup vs baseline: 12.5759x; 1.0004x over previous
import jax
import jax.numpy as jnp
import numpy as np
from jax import lax
from jax.experimental import pallas as pl
from jax.experimental.pallas import tpu as pltpu
from jax.experimental.pallas import tpu_sc as plsc

N = 10000
D = 128
NP = 10112
NC = 2
NS = 16
NW = NC * NS
BATCH = 128
NBATCH = 80
EP = NW * NBATCH * BATCH
STRIPE = NP // NS


HB = NBATCH // 2


def _agg_body(src_hbm, dst_hbm, h_hbm, zrow_hbm, out_hbm,
              acc_sh, srcv, dstv, rows0, rows1, semA, semB):
    c = lax.axis_index("c")
    s = lax.axis_index("s")
    wid = c * NS + s

    pltpu.sync_copy(zrow_hbm, acc_sh.at[pl.ds(s * STRIPE, STRIPE)])
    plsc.subcore_barrier()

    for half in range(2):
        pltpu.sync_copy(src_hbm.at[wid, pl.ds(half * HB, HB)], srcv)
        pltpu.sync_copy(dst_hbm.at[wid, pl.ds(half * HB, HB)], dstv)
        pltpu.async_copy(h_hbm.at[srcv.at[0]], rows0, semA)

        def step(jj, carry):
            j0 = 2 * jj
            j1 = j0 + 1
            pltpu.async_copy(h_hbm.at[srcv.at[j1]], rows1, semB)
            pltpu.make_async_copy(h_hbm.at[srcv.at[j0]], rows0, semA).wait()
            pltpu.sync_copy(rows0, acc_sh.at[dstv.at[j0]], add=True)

            @pl.when(j0 + 2 < HB)
            def _():
                pltpu.async_copy(h_hbm.at[srcv.at[j0 + 2]], rows0, semA)

            pltpu.make_async_copy(h_hbm.at[srcv.at[j1]], rows1, semB).wait()
            pltpu.sync_copy(rows1, acc_sh.at[dstv.at[j1]], add=True)
            return carry

        lax.fori_loop(0, HB // 2, step, 0)
    plsc.subcore_barrier()

    pltpu.sync_copy(acc_sh.at[pl.ds(s * STRIPE, STRIPE)],
                    out_hbm.at[pl.ds(c * NP + s * STRIPE, STRIPE)])


_agg = pl.kernel(
    _agg_body,
    out_type=jax.ShapeDtypeStruct((NC * NP, D), jnp.float32),
    mesh=plsc.VectorSubcoreMesh(core_axis_name="c", subcore_axis_name="s"),
    scratch_types=[
        pltpu.VMEM_SHARED((NP, D), jnp.float32),
        pltpu.VMEM((HB, BATCH), jnp.int32),
        pltpu.VMEM((HB, BATCH), jnp.int32),
        pltpu.VMEM((BATCH, D), jnp.float32),
        pltpu.VMEM((BATCH, D), jnp.float32),
        pltpu.SemaphoreType.DMA,
        pltpu.SemaphoreType.DMA,
    ],
)


def _agg_cnt_body(src_hbm, dst_hbm, h_hbm, zrow_hbm, zc_hbm, out_hbm,
                  cnt_hbm, acc_sh, cnt_sh, srcv, dstv, rows0, rows1, onesv,
                  semA, semB, semC):
    c = lax.axis_index("c")
    s = lax.axis_index("s")
    wid = c * NS + s

    pltpu.sync_copy(zrow_hbm, acc_sh.at[pl.ds(s * STRIPE, STRIPE)])

    @pl.when(s == 0)
    def _():
        pltpu.sync_copy(zc_hbm, cnt_sh)

    def fill(i, carry):
        onesv[pl.ds(i * 16, 16)] = jnp.ones((16,), jnp.float32)
        return carry

    lax.fori_loop(0, BATCH // 16, fill, 0)
    plsc.subcore_barrier()

    for half in range(2):
        pltpu.sync_copy(src_hbm.at[wid, pl.ds(half * HB, HB)], srcv)
        pltpu.sync_copy(dst_hbm.at[wid, pl.ds(half * HB, HB)], dstv)
        pltpu.async_copy(h_hbm.at[srcv.at[0]], rows0, semA)

        def step(jj, carry):
            j0 = 2 * jj
            j1 = j0 + 1
            dc0 = pltpu.async_copy(onesv, cnt_sh.at[dstv.at[j0]], semC,
                                   add=True)
            dc1 = pltpu.async_copy(onesv, cnt_sh.at[dstv.at[j1]], semC,
                                   add=True)
            pltpu.async_copy(h_hbm.at[srcv.at[j1]], rows1, semB)
            pltpu.make_async_copy(h_hbm.at[srcv.at[j0]], rows0, semA).wait()
            pltpu.sync_copy(rows0, acc_sh.at[dstv.at[j0]], add=True)

            @pl.when(j0 + 2 < HB)
            def _():
                pltpu.async_copy(h_hbm.at[srcv.at[j0 + 2]], rows0, semA)

            pltpu.make_async_copy(h_hbm.at[srcv.at[j1]], rows1, semB).wait()
            pltpu.sync_copy(rows1, acc_sh.at[dstv.at[j1]], add=True)
            dc0.wait()
            dc1.wait()
            return carry

        lax.fori_loop(0, HB // 2, step, 0)
    plsc.subcore_barrier()

    pltpu.sync_copy(acc_sh.at[pl.ds(s * STRIPE, STRIPE)],
                    out_hbm.at[pl.ds(c * NP + s * STRIPE, STRIPE)])

    @pl.when(s == 1)
    def _():
        pltpu.sync_copy(cnt_sh, cnt_hbm.at[pl.ds(c * NP, NP)])


_agg_cnt = pl.kernel(
    _agg_cnt_body,
    out_type=(jax.ShapeDtypeStruct((NC * NP, D), jnp.float32),
              jax.ShapeDtypeStruct((NC * NP,), jnp.float32)),
    mesh=plsc.VectorSubcoreMesh(core_axis_name="c", subcore_axis_name="s"),
    scratch_types=[
        pltpu.VMEM_SHARED((NP, D), jnp.float32),
        pltpu.VMEM_SHARED((NP,), jnp.float32),
        pltpu.VMEM((HB, BATCH), jnp.int32),
        pltpu.VMEM((HB, BATCH), jnp.int32),
        pltpu.VMEM((BATCH, D), jnp.float32),
        pltpu.VMEM((BATCH, D), jnp.float32),
        pltpu.VMEM((BATCH,), jnp.float32),
        pltpu.SemaphoreType.DMA,
        pltpu.SemaphoreType.DMA,
        pltpu.SemaphoreType.DMA,
    ],
)


def _dense_body(acc_ref, cnt_ref, h_ref, wl_ref, wr_ref, b_ref, g_ref,
                be_ref, out_ref):
    msum = acc_ref[:N, :] + acc_ref[NP:NP + N, :]
    mean = msum * (1.0 / jnp.maximum(cnt_ref[:, 0:1], 1.0))
    t = (jnp.dot(mean, wl_ref[:], preferred_element_type=jnp.float32)
         + jnp.dot(h_ref[:], wr_ref[:], preferred_element_type=jnp.float32)
         + b_ref[:])
    mu = jnp.mean(t, axis=0, keepdims=True)
    var = jnp.mean((t - mu) * (t - mu), axis=0, keepdims=True)
    y = (t - mu) * lax.rsqrt(var + 1e-5) * g_ref[:] + be_ref[:]
    out_ref[:] = jnp.maximum(y, 0.0)


_dense = pl.pallas_call(
    _dense_body,
    out_shape=jax.ShapeDtypeStruct((N, D), jnp.float32),
)


def _dense_mlp_body(acc_ref, cnt_ref, h_ref, wl_ref, wr_ref, b_ref, g_ref,
                    be_ref, h1_ref, w1a_ref, w1b_ref, w1c_ref, bc1_ref,
                    w2_ref, bc2_ref, out_ref):
    msum = acc_ref[:N, :] + acc_ref[NP:NP + N, :]
    mean = msum * (1.0 / jnp.maximum(cnt_ref[:, 0:1], 1.0))
    t = (jnp.dot(mean, wl_ref[:], preferred_element_type=jnp.float32)
         + jnp.dot(h_ref[:], wr_ref[:], preferred_element_type=jnp.float32)
         + b_ref[:])
    mu = jnp.mean(t, axis=0, keepdims=True)
    var = jnp.mean((t - mu) * (t - mu), axis=0, keepdims=True)
    y = (t - mu) * lax.rsqrt(var + 1e-5) * g_ref[:] + be_ref[:]
    h3 = jnp.maximum(y, 0.0)
    z = (jnp.dot(h1_ref[:], w1a_ref[:], preferred_element_type=jnp.float32)
         + jnp.dot(h_ref[:], w1b_ref[:], preferred_element_type=jnp.float32)
         + jnp.dot(h3, w1c_ref[:], preferred_element_type=jnp.float32)
         + bc1_ref[:])
    z = jnp.maximum(z, 0.0)
    out_ref[:] = (jnp.dot(z, w2_ref[:], preferred_element_type=jnp.float32)
                  + bc2_ref[:])


_dense_mlp = pl.pallas_call(
    _dense_mlp_body,
    out_shape=jax.ShapeDtypeStruct((N, 2), jnp.float32),
)

_AR = np.arange(EP - 320000, dtype=np.int32)
_PSRC = (_AR % N).reshape(NW, -1)
_PDST = (N + (_AR % 16)).reshape(NW, -1).astype(np.int32)


def kernel(x, edge_index, Wl0, Wr0, b0, g0, be0, Wl1, Wr1, b1, g1, be1,
           Wl2, Wr2, b2, g2, be2, Wc1, bc1, Wc2, bc2):
    src = edge_index[0]
    dst = edge_index[1]
    E = src.shape[0]
    src_p = jnp.concatenate([src.reshape(NW, E // NW), _PSRC],
                            axis=1).reshape(NW, NBATCH, BATCH)
    dst_p = jnp.concatenate([dst.reshape(NW, E // NW), _PDST],
                            axis=1).reshape(NW, NBATCH, BATCH)

    zrow = jnp.zeros((STRIPE, D), jnp.float32)
    zc = jnp.zeros((NP,), jnp.float32)

    acc0, cnt1d = _agg_cnt(src_p, dst_p, x, zrow, zc)
    cnt = jnp.broadcast_to((cnt1d[:N] + cnt1d[NP:NP + N])[:, None], (N, 16))

    b0r, g0r, be0r = b0[None, :], g0[None, :], be0[None, :]
    b1r, g1r, be1r = b1[None, :], g1[None, :], be1[None, :]
    b2r, g2r, be2r = b2[None, :], g2[None, :], be2[None, :]

    h1 = _dense(acc0, cnt, x, Wl0, Wr0, b0r, g0r, be0r)
    acc1 = _agg(src_p, dst_p, h1, zrow)
    h2 = _dense(acc1, cnt, h1, Wl1, Wr1, b1r, g1r, be1r)
    acc2 = _agg(src_p, dst_p, h2, zrow)

    w1a, w1b, w1c = Wc1[:128], Wc1[128:256], Wc1[256:]
    return _dense_mlp(acc2, cnt, h2, Wl2, Wr2, b2r, g2r, be2r,
                      h1, w1a, w1b, w1c, bc1[None, :], Wc2, bc2[None, :])


# speedup vs baseline: 12.7209x; 1.0115x over previous
import jax
import jax.numpy as jnp
import numpy as np
from jax import lax
from jax.experimental import pallas as pl
from jax.experimental.pallas import tpu as pltpu
from jax.experimental.pallas import tpu_sc as plsc

N = 10000
D = 128
NP = 10112
NC = 2
NS = 16
NW = NC * NS
BATCH = 128
NBATCH = 80
EP = NW * NBATCH * BATCH
STRIPE = NP // NS


HB = NBATCH // 2


def _agg_body(src_hbm, dst_hbm, h_hbm, zrow_hbm, out_hbm,
              acc_sh, srcv, dstv, rows0, rows1, semA, semB):
    c = lax.axis_index("c")
    s = lax.axis_index("s")
    wid = c * NS + s

    pltpu.sync_copy(zrow_hbm, acc_sh.at[pl.ds(s * STRIPE, STRIPE)])
    plsc.subcore_barrier()

    for half in range(2):
        pltpu.sync_copy(src_hbm.at[wid, pl.ds(half * HB, HB)], srcv)
        pltpu.sync_copy(dst_hbm.at[wid, pl.ds(half * HB, HB)], dstv)
        pltpu.async_copy(h_hbm.at[srcv.at[0]], rows0, semA)

        def step(jj, carry):
            j0 = 2 * jj
            j1 = j0 + 1
            pltpu.async_copy(h_hbm.at[srcv.at[j1]], rows1, semB)
            pltpu.make_async_copy(h_hbm.at[srcv.at[j0]], rows0, semA).wait()
            pltpu.sync_copy(rows0, acc_sh.at[dstv.at[j0]], add=True)

            @pl.when(j0 + 2 < HB)
            def _():
                pltpu.async_copy(h_hbm.at[srcv.at[j0 + 2]], rows0, semA)

            pltpu.make_async_copy(h_hbm.at[srcv.at[j1]], rows1, semB).wait()
            pltpu.sync_copy(rows1, acc_sh.at[dstv.at[j1]], add=True)
            return carry

        lax.fori_loop(0, HB // 2, step, 0)
    plsc.subcore_barrier()

    pltpu.sync_copy(acc_sh.at[pl.ds(s * STRIPE, STRIPE)],
                    out_hbm.at[pl.ds(c * NP + s * STRIPE, STRIPE)])


_agg = pl.kernel(
    _agg_body,
    out_type=jax.ShapeDtypeStruct((NC * NP, D), jnp.float32),
    mesh=plsc.VectorSubcoreMesh(core_axis_name="c", subcore_axis_name="s"),
    scratch_types=[
        pltpu.VMEM_SHARED((NP, D), jnp.float32),
        pltpu.VMEM((HB, BATCH), jnp.int32),
        pltpu.VMEM((HB, BATCH), jnp.int32),
        pltpu.VMEM((BATCH, D), jnp.float32),
        pltpu.VMEM((BATCH, D), jnp.float32),
        pltpu.SemaphoreType.DMA,
        pltpu.SemaphoreType.DMA,
    ],
)


def _agg_cnt_body(src_hbm, dst_hbm, h_hbm, zrow_hbm, zc_hbm, out_hbm,
                  cnt_hbm, acc_sh, cnt_sh, srcv, dstv, rows0, rows1, onesv,
                  semA, semB, semC):
    c = lax.axis_index("c")
    s = lax.axis_index("s")
    wid = c * NS + s

    pltpu.sync_copy(zrow_hbm, acc_sh.at[pl.ds(s * STRIPE, STRIPE)])

    @pl.when(s == 0)
    def _():
        pltpu.sync_copy(zc_hbm, cnt_sh)

    def fill(i, carry):
        onesv[pl.ds(i * 16, 16)] = jnp.ones((16,), jnp.float32)
        return carry

    lax.fori_loop(0, BATCH // 16, fill, 0)
    plsc.subcore_barrier()

    for half in range(2):
        pltpu.sync_copy(src_hbm.at[wid, pl.ds(half * HB, HB)], srcv)
        pltpu.sync_copy(dst_hbm.at[wid, pl.ds(half * HB, HB)], dstv)
        pltpu.async_copy(h_hbm.at[srcv.at[0]], rows0, semA)

        def step(jj, carry):
            j0 = 2 * jj
            j1 = j0 + 1
            dc0 = pltpu.async_copy(onesv, cnt_sh.at[dstv.at[j0]], semC,
                                   add=True)
            dc1 = pltpu.async_copy(onesv, cnt_sh.at[dstv.at[j1]], semC,
                                   add=True)
            pltpu.async_copy(h_hbm.at[srcv.at[j1]], rows1, semB)
            pltpu.make_async_copy(h_hbm.at[srcv.at[j0]], rows0, semA).wait()
            pltpu.sync_copy(rows0, acc_sh.at[dstv.at[j0]], add=True)

            @pl.when(j0 + 2 < HB)
            def _():
                pltpu.async_copy(h_hbm.at[srcv.at[j0 + 2]], rows0, semA)

            pltpu.make_async_copy(h_hbm.at[srcv.at[j1]], rows1, semB).wait()
            pltpu.sync_copy(rows1, acc_sh.at[dstv.at[j1]], add=True)
            dc0.wait()
            dc1.wait()
            return carry

        lax.fori_loop(0, HB // 2, step, 0)
    plsc.subcore_barrier()

    pltpu.sync_copy(acc_sh.at[pl.ds(s * STRIPE, STRIPE)],
                    out_hbm.at[pl.ds(c * NP + s * STRIPE, STRIPE)])

    @pl.when(s == 1)
    def _():
        pltpu.sync_copy(cnt_sh, cnt_hbm.at[pl.ds(c * NP, NP)])


_agg_cnt = pl.kernel(
    _agg_cnt_body,
    out_type=(jax.ShapeDtypeStruct((NC * NP, D), jnp.float32),
              jax.ShapeDtypeStruct((NC * NP,), jnp.float32)),
    mesh=plsc.VectorSubcoreMesh(core_axis_name="c", subcore_axis_name="s"),
    scratch_types=[
        pltpu.VMEM_SHARED((NP, D), jnp.float32),
        pltpu.VMEM_SHARED((NP,), jnp.float32),
        pltpu.VMEM((HB, BATCH), jnp.int32),
        pltpu.VMEM((HB, BATCH), jnp.int32),
        pltpu.VMEM((BATCH, D), jnp.float32),
        pltpu.VMEM((BATCH, D), jnp.float32),
        pltpu.VMEM((BATCH,), jnp.float32),
        pltpu.SemaphoreType.DMA,
        pltpu.SemaphoreType.DMA,
        pltpu.SemaphoreType.DMA,
    ],
)


def _dense_body(acc_ref, cnt_ref, h_ref, wl_ref, wr_ref, b_ref, g_ref,
                be_ref, out_ref):
    msum = acc_ref[:N, :] + acc_ref[NP:NP + N, :]
    mean = msum * (1.0 / jnp.maximum(cnt_ref[:, 0:1], 1.0))
    t = (jnp.dot(mean, wl_ref[:], preferred_element_type=jnp.float32)
         + jnp.dot(h_ref[:], wr_ref[:], preferred_element_type=jnp.float32)
         + b_ref[:])
    mu = jnp.mean(t, axis=0, keepdims=True)
    var = jnp.mean((t - mu) * (t - mu), axis=0, keepdims=True)
    y = (t - mu) * lax.rsqrt(var + 1e-5) * g_ref[:] + be_ref[:]
    out_ref[:] = jnp.maximum(y, 0.0)


_dense = pl.pallas_call(
    _dense_body,
    out_shape=jax.ShapeDtypeStruct((N, D), jnp.float32),
)


def _dense_mlp_body(acc_ref, cnt_ref, h_ref, wl_ref, wr_ref, b_ref, g_ref,
                    be_ref, h1_ref, w1a_ref, w1b_ref, w1c_ref, bc1_ref,
                    w2_ref, bc2_ref, out_ref):
    msum = acc_ref[:N, :] + acc_ref[NP:NP + N, :]
    mean = msum * (1.0 / jnp.maximum(cnt_ref[:, 0:1], 1.0))
    t = (jnp.dot(mean, wl_ref[:], preferred_element_type=jnp.float32)
         + jnp.dot(h_ref[:], wr_ref[:], preferred_element_type=jnp.float32)
         + b_ref[:])
    mu = jnp.mean(t, axis=0, keepdims=True)
    var = jnp.mean((t - mu) * (t - mu), axis=0, keepdims=True)
    y = (t - mu) * lax.rsqrt(var + 1e-5) * g_ref[:] + be_ref[:]
    h3 = jnp.maximum(y, 0.0)
    z = (jnp.dot(h1_ref[:], w1a_ref[:], preferred_element_type=jnp.float32)
         + jnp.dot(h_ref[:], w1b_ref[:], preferred_element_type=jnp.float32)
         + jnp.dot(h3, w1c_ref[:], preferred_element_type=jnp.float32)
         + bc1_ref[:])
    z = jnp.maximum(z, 0.0)
    out_ref[:] = (jnp.dot(z, w2_ref[:], preferred_element_type=jnp.float32)
                  + bc2_ref[:])


_dense_mlp = pl.pallas_call(
    _dense_mlp_body,
    out_shape=jax.ShapeDtypeStruct((N, 2), jnp.float32),
)

_AR = np.arange(EP - 320000, dtype=np.int32)
_PSRC = (_AR % N).reshape(-1, BATCH)
_PDST = (N + (_AR % 16)).reshape(-1, BATCH).astype(np.int32)


def kernel(x, edge_index, Wl0, Wr0, b0, g0, be0, Wl1, Wr1, b1, g1, be1,
           Wl2, Wr2, b2, g2, be2, Wc1, bc1, Wc2, bc2):
    src = edge_index[0]
    dst = edge_index[1]
    E = src.shape[0]
    src_p = jnp.concatenate([src.reshape(E // BATCH, BATCH), _PSRC],
                            axis=0).reshape(NW, NBATCH, BATCH)
    dst_p = jnp.concatenate([dst.reshape(E // BATCH, BATCH), _PDST],
                            axis=0).reshape(NW, NBATCH, BATCH)

    zrow = jnp.zeros((STRIPE, D), jnp.float32)
    zc = jnp.zeros((NP,), jnp.float32)

    acc0, cnt1d = _agg_cnt(src_p, dst_p, x, zrow, zc)
    cnt = jnp.broadcast_to((cnt1d[:N] + cnt1d[NP:NP + N])[:, None], (N, 16))

    b0r, g0r, be0r = b0[None, :], g0[None, :], be0[None, :]
    b1r, g1r, be1r = b1[None, :], g1[None, :], be1[None, :]
    b2r, g2r, be2r = b2[None, :], g2[None, :], be2[None, :]

    h1 = _dense(acc0, cnt, x, Wl0, Wr0, b0r, g0r, be0r)
    acc1 = _agg(src_p, dst_p, h1, zrow)
    h2 = _dense(acc1, cnt, h1, Wl1, Wr1, b1r, g1r, be1r)
    acc2 = _agg(src_p, dst_p, h2, zrow)

    w1a, w1b, w1c = Wc1[:128], Wc1[128:256], Wc1[256:]
    return _dense_mlp(acc2, cnt, h2, Wl2, Wr2, b2r, g2r, be2r,
                      h1, w1a, w1b, w1c, bc1[None, :], Wc2, bc2[None, :])
